# Initial kernel scaffold; baseline (speedup 1.0000x reference)
import jax
import jax.numpy as jnp
from jax.experimental import pallas as pl


def kernel(x, edge_index, Wl1, bl1, Wr1, Wl2, bl2, Wr2, Wl3, bl3, Wr3, Wp1, bp1, Wp2, bp2, Wc, bc):
    raise NotImplementedError("write your pallas kernel here")


# speedup vs baseline: 2.4718x; 2.4718x over previous
import functools

import jax
import jax.numpy as jnp
from jax import lax
from jax.experimental import pallas as pl
from jax.experimental.pallas import tpu as pltpu
from jax.experimental.pallas import tpu_sc as plsc

N = 10000
E = 320000
D = 128
NPAD = 10240
NW = 32
EPW = 10240
EPAD = EPW * NW
CH = 128
NCH = EPW // CH
RPS = NPAD // 16

_f32 = jnp.float32

_mesh = plsc.VectorSubcoreMesh(core_axis_name="c", subcore_axis_name="s")


@functools.partial(
    pl.kernel,
    out_type=[
        jax.ShapeDtypeStruct((2, NPAD, D), _f32),
    ],
    scratch_types=[
        pltpu.VMEM((CH,), jnp.int32),
        pltpu.VMEM((CH,), jnp.int32),
        pltpu.VMEM((CH, D), _f32),
        pltpu.VMEM_SHARED((NPAD, D), _f32),
        pltpu.SemaphoreType.DMA,
    ],
    mesh=_mesh,
)
def _sc_segsum(table, src, dst, zrows, parts,
               sidx, didx, rows, acc, sem):
    cid = lax.axis_index("c")
    sid = lax.axis_index("s")
    wid = cid * 16 + sid
    base = wid * EPW

    pltpu.sync_copy(zrows, acc.at[pl.ds(sid * RPS, RPS), :])
    plsc.subcore_barrier()

    def _chunk(i, carry):
        b = base + i * CH
        pltpu.sync_copy(src.at[pl.ds(b, CH)], sidx)
        pltpu.sync_copy(dst.at[pl.ds(b, CH)], didx)
        pltpu.async_copy(table.at[sidx], rows, sem).wait()
        pltpu.sync_copy(rows, acc.at[didx], add=True)
        return carry
    lax.fori_loop(0, NCH, _chunk, 0)

    plsc.subcore_barrier()
    pltpu.sync_copy(acc.at[pl.ds(sid * RPS, RPS), :],
                    parts.at[cid, pl.ds(sid * RPS, RPS), :])


@functools.partial(
    pl.kernel,
    out_type=[
        jax.ShapeDtypeStruct((2, NPAD, D), _f32),
    ],
    scratch_types=[
        pltpu.VMEM((CH,), jnp.int32),
        pltpu.VMEM((CH, D), _f32),
        pltpu.VMEM_SHARED((NPAD, D), _f32),
        pltpu.SemaphoreType.DMA,
    ],
    mesh=_mesh,
)
def _sc_cnt(dst, zrows, ones_rows, cnt_parts,
            didx, onesb, cacc, sem):
    cid = lax.axis_index("c")
    sid = lax.axis_index("s")
    wid = cid * 16 + sid
    base = wid * EPW

    pltpu.sync_copy(ones_rows, onesb)
    pltpu.sync_copy(zrows, cacc.at[pl.ds(sid * RPS, RPS), :])
    plsc.subcore_barrier()

    def _chunk(i, carry):
        b = base + i * CH
        pltpu.sync_copy(dst.at[pl.ds(b, CH)], didx)
        pltpu.sync_copy(onesb, cacc.at[didx], add=True)
        return carry
    lax.fori_loop(0, NCH, _chunk, 0)

    plsc.subcore_barrier()
    pltpu.sync_copy(cacc.at[pl.ds(sid * RPS, RPS), :],
                    cnt_parts.at[cid, pl.ds(sid * RPS, RPS), :])


@functools.partial(
    pl.kernel,
    out_type=[jax.ShapeDtypeStruct((EPAD * 16,), _f32)],
    scratch_types=[
        pltpu.VMEM((CH,), jnp.int32),
        pltpu.VMEM((CH,), jnp.int32),
        pltpu.VMEM((CH, D), _f32),
        pltpu.VMEM((CH, D), _f32),
        pltpu.VMEM((D,), _f32),
        pltpu.VMEM((CH * 16,), _f32),
        pltpu.SemaphoreType.DMA,
    ],
    mesh=_mesh,
)
def _sc_edge(A, B, src, dst, wp2, tsum,
             sidx, didx, rowsA, rowsB, wvec, outb, sem):
    cid = lax.axis_index("c")
    sid = lax.axis_index("s")
    wid = cid * 16 + sid
    base = wid * EPW

    pltpu.sync_copy(wp2, wvec)
    wl = [wvec[pl.ds(j * 16, 16)] for j in range(D // 16)]

    def _chunk(i, carry):
        b = base + i * CH
        pltpu.sync_copy(src.at[pl.ds(b, CH)], sidx)
        pltpu.sync_copy(dst.at[pl.ds(b, CH)], didx)
        cpA = pltpu.async_copy(A.at[sidx], rowsA, sem)
        cpB = pltpu.async_copy(B.at[didx], rowsB, sem)
        cpA.wait()
        cpB.wait()

        def _edge(e, carry2):
            t = jnp.zeros((16,), _f32)
            for j in range(D // 16):
                va = rowsA[e, pl.ds(j * 16, 16)]
                vb = rowsB[e, pl.ds(j * 16, 16)]
                t = t + jnp.maximum(va + vb, 0.0) * wl[j]
            outb[pl.ds(e * 16, 16)] = t
            return carry2
        lax.fori_loop(0, CH, _edge, 0)

        pltpu.sync_copy(outb, tsum.at[pl.ds(b * 16, CH * 16)])
        return carry
    lax.fori_loop(0, NCH, _chunk, 0)


def _mmT(a, w):
    return lax.dot_general(a, w, (((1,), (1,)), ((), ())),
                           preferred_element_type=_f32)


def _tc_inv_body(cnt_ref, inv_ref):
    cnt = cnt_ref[0, :, 0] + cnt_ref[1, :, 0]
    inv_ref[...] = (1.0 / jnp.maximum(cnt, 1.0))[:, None]


def _tc_pre_body(x_ref, wl_ref, wr_ref, g_ref, r_ref):
    x = x_ref[...]
    g_ref[...] = _mmT(x, wl_ref[...])
    r_ref[...] = _mmT(x, wr_ref[...])


def _tc_mid_body(parts_ref, inv_ref, r_ref, bl_ref, wln_ref, wrn_ref,
                 g_ref, rn_ref):
    s = parts_ref[0] + parts_ref[1]
    h = jnp.maximum(s * inv_ref[...] + bl_ref[...] + r_ref[...], 0.0)
    g_ref[...] = _mmT(h, wln_ref[...])
    rn_ref[...] = _mmT(h, wrn_ref[...])


def _tc_fin_body(parts_ref, inv_ref, r_ref, bl_ref, wc_ref, bc_ref,
                 wps_ref, wpd_ref, bp1_ref, ne_ref, sup_ref, a_ref, b_ref):
    s = parts_ref[0] + parts_ref[1]
    ne = s * inv_ref[...] + bl_ref[...] + r_ref[...]
    ne_ref[...] = ne
    sup_ref[...] = _mmT(ne, wc_ref[...]) + bc_ref[...]
    a_ref[...] = _mmT(ne, wps_ref[...]) + bp1_ref[...]
    b_ref[...] = _mmT(ne, wpd_ref[...])


def _tc_edge_fin_body(t_ref, m_ref, out_ref):
    out_ref[...] = lax.dot_general(t_ref[...], m_ref[...],
                                   (((1,), (0,)), ((), ())),
                                   preferred_element_type=_f32)


def _tc(body, out_shape, *args):
    return pl.pallas_call(body, out_shape=out_shape)(*args)


def kernel(x, edge_index, Wl1, bl1, Wr1, Wl2, bl2, Wr2, Wl3, bl3, Wr3,
           Wp1, bp1, Wp2, bp2, Wc, bc):
    src = edge_index[0]
    dst = edge_index[1]
    src_p = jnp.concatenate([src, jnp.zeros((EPAD - E,), jnp.int32)])
    dst_p = jnp.concatenate([dst, jnp.full((EPAD - E,), N + 100, jnp.int32)])
    x_p = jnp.pad(x, ((0, NPAD - N), (0, 0)))
    zrows = jnp.zeros((RPS, D), _f32)
    ones_rows = jnp.ones((CH, D), _f32)
    Wc_p = jnp.pad(Wc, ((0, D - Wc.shape[0]), (0, 0)))
    bc_p = jnp.pad(bc, (0, D - bc.shape[0])).reshape(1, D)
    Wp1s = Wp1[:, :D]
    Wp1d = Wp1[:, D:]

    sds = jax.ShapeDtypeStruct
    g1, r1 = _tc(_tc_pre_body,
                 [sds((NPAD, D), _f32), sds((NPAD, D), _f32)],
                 x_p, Wl1, Wr1)
    (cnt_parts,) = _sc_cnt(dst_p, zrows, ones_rows)
    (parts1,) = _sc_segsum(g1, src_p, dst_p, zrows)
    inv = _tc(_tc_inv_body, sds((NPAD, 1), _f32), cnt_parts)

    g2, r2 = _tc(_tc_mid_body,
                 [sds((NPAD, D), _f32), sds((NPAD, D), _f32)],
                 parts1, inv, r1, bl1.reshape(1, D), Wl2, Wr2)
    (parts2,) = _sc_segsum(g2, src_p, dst_p, zrows)

    g3, r3 = _tc(_tc_mid_body,
                 [sds((NPAD, D), _f32), sds((NPAD, D), _f32)],
                 parts2, inv, r2, bl2.reshape(1, D), Wl3, Wr3)
    (parts3,) = _sc_segsum(g3, src_p, dst_p, zrows)

    ne, sup, Ab, Bb = _tc(
        _tc_fin_body,
        [sds((NPAD, D), _f32), sds((NPAD, D), _f32),
         sds((NPAD, D), _f32), sds((NPAD, D), _f32)],
        parts3, inv, r3, bl3.reshape(1, D), Wc_p, bc_p,
        Wp1s, Wp1d, bp1.reshape(1, D))

    (tsum,) = _sc_edge(Ab, Bb, src_p, dst_p, Wp2.reshape(D))
    msel = jnp.repeat(jnp.eye(8, dtype=_f32), 16, axis=0)
    flow = _tc(_tc_edge_fin_body, sds((EPAD // 8, 8), _f32),
               tsum.reshape(EPAD // 8, D), msel)

    node_embeddings = ne[:N]
    carbon_flows = flow.reshape(EPAD, 1)[:E] + bp2
    supplier_classes = sup[:N, :4]
    return (node_embeddings, carbon_flows, supplier_classes)


# speedup vs baseline: 3.3384x; 1.3506x over previous
import functools

import jax
import jax.numpy as jnp
from jax import lax
from jax.experimental import pallas as pl
from jax.experimental.pallas import tpu as pltpu
from jax.experimental.pallas import tpu_sc as plsc

N = 10000
E = 320000
D = 128
NPAD = 10240
NW = 32
EPW = 10240
EPAD = EPW * NW
CH = 128
NCH = EPW // CH
RPS = NPAD // 16
NQ = RPS // CH

_f32 = jnp.float32

_mesh = plsc.VectorSubcoreMesh(core_axis_name="c", subcore_axis_name="s")


CG = 40
NCG = EPW // CG
IB = 32
NB = NCG // IB
NQ2 = RPS // CG


@functools.partial(
    pl.kernel,
    out_type=[
        jax.ShapeDtypeStruct((2, NPAD, D), _f32),
    ],
    scratch_types=[
        pltpu.VMEM((IB, CG), jnp.int32),
        pltpu.VMEM((IB, CG), jnp.int32),
        pltpu.VMEM((2, CG, D), _f32),
        pltpu.VMEM_SHARED((NPAD, D), _f32),
        pltpu.SemaphoreType.DMA,
        pltpu.SemaphoreType.DMA,
    ],
    mesh=_mesh,
)
def _sc_segmean(table, src3, dst3, zrows, parts,
                sidx2, didx2, rows2,
                acc, gsem, ssem):
    cid = lax.axis_index("c")
    sid = lax.axis_index("s")
    wid = cid * 16 + sid

    pltpu.sync_copy(zrows, acc.at[pl.ds(sid * RPS, RPS), :])

    plsc.subcore_barrier()

    def _fire_gather(i, b):
        pltpu.async_copy(table.at[sidx2.at[i]], rows2.at[b], gsem)

    def _wait_gather(i, b):
        pltpu.make_async_copy(table.at[sidx2.at[i]], rows2.at[b], gsem).wait()

    def _fire_scat(i, b):
        pltpu.async_copy(rows2.at[b], acc.at[didx2.at[i]], ssem, add=True)

    def _wait_scat(i, b):
        pltpu.make_async_copy(rows2.at[b], acc.at[didx2.at[i]], ssem).wait()

    for batch in range(NB):
        pltpu.sync_copy(src3.at[wid, pl.ds(batch * IB, IB)], sidx2)
        pltpu.sync_copy(dst3.at[wid, pl.ds(batch * IB, IB)], didx2)
        for b in range(2):
            _fire_gather(b, b)

        def _inner(o, carry):
            for b in range(2):
                i = o * 2 + b
                _wait_gather(i, b)
                _fire_scat(i, b)
                _wait_scat(i, b)
                _fire_gather(i + 2, b)
            return carry
        lax.fori_loop(0, IB // 2 - 1, _inner, 0)

        for b in range(2):
            i = IB - 2 + b
            _wait_gather(i, b)
            _fire_scat(i, b)
            _wait_scat(i, b)

    plsc.subcore_barrier()

    pltpu.sync_copy(acc.at[pl.ds(sid * RPS, RPS), :],
                    parts.at[cid, pl.ds(sid * RPS, RPS), :])


@functools.partial(
    pl.kernel,
    out_type=[
        jax.ShapeDtypeStruct((2, NPAD, D), _f32),
    ],
    scratch_types=[
        pltpu.VMEM((NCH, CH), jnp.int32),
        pltpu.VMEM((CH, D), _f32),
        pltpu.VMEM_SHARED((NPAD, D), _f32),
        pltpu.SemaphoreType.DMA,
    ],
    mesh=_mesh,
)
def _sc_cnt(dst3, zrows, cnt_parts, didx2, onesb, cacc, ssem):
    cid = lax.axis_index("c")
    sid = lax.axis_index("s")
    wid = cid * 16 + sid

    pltpu.sync_copy(dst3.at[wid], didx2)
    pltpu.sync_copy(zrows, cacc.at[pl.ds(sid * RPS, RPS), :])

    one16 = jnp.full((16,), 1.0, _f32)

    def _fill(r, carry):
        for j in range(D // 16):
            onesb[r, pl.ds(j * 16, 16)] = one16
        return carry
    lax.fori_loop(0, CH, _fill, 0)
    plsc.subcore_barrier()

    def _fire(i):
        pltpu.async_copy(onesb, cacc.at[didx2.at[i]], ssem, add=True)

    def _drain(i):
        pltpu.make_async_copy(onesb, cacc.at[didx2.at[i]], ssem).wait()

    for b in range(8):
        _fire(b)

    def _outer(o, carry):
        for b in range(8):
            i = o * 8 + b
            _drain(i)
            _fire(i + 8)
        return carry
    lax.fori_loop(0, NCH // 8 - 1, _outer, 0)

    for b in range(8):
        _drain(NCH - 8 + b)

    plsc.subcore_barrier()
    pltpu.sync_copy(cacc.at[pl.ds(sid * RPS, RPS), :],
                    cnt_parts.at[cid, pl.ds(sid * RPS, RPS), :])


@functools.partial(
    pl.kernel,
    out_type=[jax.ShapeDtypeStruct((EPAD * 16,), _f32)],
    scratch_types=[
        pltpu.VMEM((NCH, CH), jnp.int32),
        pltpu.VMEM((NCH, CH), jnp.int32),
        pltpu.VMEM((2, CH, D), _f32),
        pltpu.VMEM((2, CH, D), _f32),
        pltpu.VMEM((D,), _f32),
        pltpu.VMEM((CH * 16,), _f32),
        pltpu.SemaphoreType.DMA,
    ],
    mesh=_mesh,
)
def _sc_edge(A, B, src3, dst3, wp2, tsum,
             sidx2, didx2, rA, rB, wvec, outb, gsem):
    cid = lax.axis_index("c")
    sid = lax.axis_index("s")
    wid = cid * 16 + sid
    base = wid * EPW

    pltpu.sync_copy(src3.at[wid], sidx2)
    pltpu.sync_copy(dst3.at[wid], didx2)
    pltpu.sync_copy(wp2, wvec)
    wl = [wvec[pl.ds(j * 16, 16)] for j in range(D // 16)]

    def _fire(i, k):
        pltpu.async_copy(A.at[sidx2.at[i]], rA.at[k], gsem)
        pltpu.async_copy(B.at[didx2.at[i]], rB.at[k], gsem)

    def _wait(i, k):
        pltpu.make_async_copy(A.at[sidx2.at[i]], rA.at[k], gsem).wait()
        pltpu.make_async_copy(B.at[didx2.at[i]], rB.at[k], gsem).wait()

    def _compute(i, k):
        def _edge(e, carry2):
            t = jnp.zeros((16,), _f32)
            for j in range(D // 16):
                va = rA[k, e, pl.ds(j * 16, 16)]
                vb = rB[k, e, pl.ds(j * 16, 16)]
                t = t + jnp.maximum(va + vb, 0.0) * wl[j]
            outb[pl.ds(e * 16, 16)] = t
            return carry2
        lax.fori_loop(0, CH, _edge, 0)
        pltpu.sync_copy(outb, tsum.at[pl.ds((base + i * CH) * 16, CH * 16)])

    _fire(0, 0)
    _fire(1, 1)

    def _outer(o, carry):
        for k in range(2):
            i = o * 2 + k
            _wait(i, k)
            _compute(i, k)
            _fire(i + 2, k)
        return carry
    lax.fori_loop(0, NCH // 2 - 1, _outer, 0)

    for k in range(2):
        i = NCH - 2 + k
        _wait(i, k)
        _compute(i, k)


def _mmT(a, w):
    return lax.dot_general(a, w, (((1,), (1,)), ((), ())),
                           preferred_element_type=_f32)


def _tc_inv_body(cnt_ref, inv_ref):
    cnt = cnt_ref[0, :, 0] + cnt_ref[1, :, 0]
    inv_ref[...] = (1.0 / jnp.maximum(cnt, 1.0))[:, None]


def _tc_pre_body(x_ref, wl_ref, wr_ref, g_ref, r_ref):
    x = x_ref[...]
    g_ref[...] = _mmT(x, wl_ref[...])
    r_ref[...] = _mmT(x, wr_ref[...])


def _tc_mid_body(parts_ref, inv_ref, r_ref, bl_ref, wln_ref, wrn_ref,
                 g_ref, rn_ref):
    h = jnp.maximum((parts_ref[0] + parts_ref[1]) * inv_ref[...]
                    + bl_ref[...] + r_ref[...], 0.0)
    g_ref[...] = _mmT(h, wln_ref[...])
    rn_ref[...] = _mmT(h, wrn_ref[...])


def _tc_fin_body(parts_ref, inv_ref, r_ref, bl_ref, wc_ref, bc_ref,
                 wps_ref, wpd_ref, bp1_ref, ne_ref, sup_ref, a_ref, b_ref):
    ne = (parts_ref[0] + parts_ref[1]) * inv_ref[...] + bl_ref[...] + r_ref[...]
    ne_ref[...] = ne
    sup_ref[...] = _mmT(ne, wc_ref[...]) + bc_ref[...]
    a_ref[...] = _mmT(ne, wps_ref[...]) + bp1_ref[...]
    b_ref[...] = _mmT(ne, wpd_ref[...])


def _tc_edge_fin_body(t_ref, m_ref, out_ref):
    out_ref[...] = lax.dot_general(t_ref[...], m_ref[...],
                                   (((1,), (0,)), ((), ())),
                                   preferred_element_type=_f32)


def _tc(body, out_shape, *args):
    return pl.pallas_call(body, out_shape=out_shape)(*args)


def kernel(x, edge_index, Wl1, bl1, Wr1, Wl2, bl2, Wr2, Wl3, bl3, Wr3,
           Wp1, bp1, Wp2, bp2, Wc, bc):
    src = edge_index[0]
    dst = edge_index[1]
    src_p = jnp.concatenate([src, jnp.zeros((EPAD - E,), jnp.int32)])
    dst_p = jnp.concatenate([dst, jnp.full((EPAD - E,), N + 100, jnp.int32)])
    src3 = src_p.reshape(NW, NCH, CH)
    dst3 = dst_p.reshape(NW, NCH, CH)
    src3g = src_p.reshape(NW, NCG, CG)
    dst3g = dst_p.reshape(NW, NCG, CG)
    x_p = jnp.pad(x, ((0, NPAD - N), (0, 0)))
    zrows = jnp.zeros((RPS, D), _f32)
    Wc_p = jnp.pad(Wc, ((0, D - Wc.shape[0]), (0, 0)))
    bc_p = jnp.pad(bc, (0, D - bc.shape[0])).reshape(1, D)
    Wp1s = Wp1[:, :D]
    Wp1d = Wp1[:, D:]

    sds = jax.ShapeDtypeStruct
    g1, r1 = _tc(_tc_pre_body,
                 [sds((NPAD, D), _f32), sds((NPAD, D), _f32)],
                 x_p, Wl1, Wr1)
    (parts1,) = _sc_segmean(g1, src3g, dst3g, zrows)
    (cnt_parts,) = _sc_cnt(dst3, zrows)
    inv = _tc(_tc_inv_body, sds((NPAD, 1), _f32), cnt_parts)

    g2, r2 = _tc(_tc_mid_body,
                 [sds((NPAD, D), _f32), sds((NPAD, D), _f32)],
                 parts1, inv, r1, bl1.reshape(1, D), Wl2, Wr2)
    (parts2,) = _sc_segmean(g2, src3g, dst3g, zrows)

    g3, r3 = _tc(_tc_mid_body,
                 [sds((NPAD, D), _f32), sds((NPAD, D), _f32)],
                 parts2, inv, r2, bl2.reshape(1, D), Wl3, Wr3)
    (parts3,) = _sc_segmean(g3, src3g, dst3g, zrows)

    ne, sup, Ab, Bb = _tc(
        _tc_fin_body,
        [sds((NPAD, D), _f32), sds((NPAD, D), _f32),
         sds((NPAD, D), _f32), sds((NPAD, D), _f32)],
        parts3, inv, r3, bl3.reshape(1, D), Wc_p, bc_p,
        Wp1s, Wp1d, bp1.reshape(1, D))

    (tsum,) = _sc_edge(Ab, Bb, src3, dst3, Wp2.reshape(D))
    msel = jnp.repeat(jnp.eye(8, dtype=_f32), 16, axis=0)
    flow = _tc(_tc_edge_fin_body, sds((EPAD // 8, 8), _f32),
               tsum.reshape(EPAD // 8, D), msel)

    node_embeddings = ne[:N]
    carbon_flows = flow.reshape(EPAD, 1)[:E] + bp2
    supplier_classes = sup[:N, :4]
    return (node_embeddings, carbon_flows, supplier_classes)
